# baseline (speedup 1.0000x reference)
import functools

import jax
import jax.numpy as jnp
from jax import lax
from jax.experimental import pallas as pl
from jax.experimental.pallas import tpu as pltpu
from jax.experimental.pallas import tpu_sc as plsc

_BATCH = 16384
_HIST = 50
_DIM = 64
_VOCAB = 1000000
_B = _BATCH * _HIST

_NC = 2
_NS = 16
_NW = _NC * _NS
_BPW = _B // _NW
_C = 128
_NCHUNK = _BPW // _C

_mesh = plsc.VectorSubcoreMesh(core_axis_name="c", subcore_axis_name="s")


@functools.partial(
    pl.kernel,
    mesh=_mesh,
    out_type=jax.ShapeDtypeStruct((_B, _DIM), jnp.float32),
    scratch_types=[
        pltpu.VMEM((_BPW,), jnp.int32),
        pltpu.VMEM((4, _C, 128), jnp.float32),
        pltpu.VMEM((2, _C, _DIM), jnp.float32),
    ]
    + [pltpu.SemaphoreType.DMA] * 6,
)
def _gather_kernel(idx_hbm, tabp_hbm, out_hbm, idx_all, pairs, rows,
                   gsem0, gsem1, gsem2, gsem3, wsem0, wsem1):
    wid = lax.axis_index("s") * _NC + lax.axis_index("c")
    base = wid * _BPW
    gsem = (gsem0, gsem1, gsem2, gsem3)
    wsem = (wsem0, wsem1)
    pltpu.sync_copy(idx_hbm.at[pl.ds(base, _BPW)], idx_all)

    def fire_gather(c, s):
        src = tabp_hbm.at[idx_all.at[pl.ds(c * _C, _C)]]
        pltpu.async_copy(src, pairs.at[s], gsem[s])

    def wait_gather(s):
        src = tabp_hbm.at[idx_all.at[pl.ds(0, _C)]]
        pltpu.make_async_copy(src, pairs.at[s], gsem[s]).wait()

    def compact(s, w):
        def gbody(g, carry):
            for u in range(16):
                j = g * 16 + u
                for k in range(_DIM // 16):
                    rows[w, j, pl.ds(k * 16, 16)] = (
                        pairs[s, j, pl.ds(k * 16, 16)]
                    )
            return carry

        lax.fori_loop(0, _C // 16, gbody, 0)

    def start_write(c, w):
        dst = out_hbm.at[pl.ds(base + c * _C, _C)]
        pltpu.async_copy(rows.at[w], dst, wsem[w])

    def wait_write(w):
        dst = out_hbm.at[pl.ds(base, _C)]
        pltpu.make_async_copy(rows.at[w], dst, wsem[w]).wait()

    fire_gather(0, 0)
    fire_gather(1, 1)

    def body(i, carry):
        for s in range(4):
            c = 4 * i + s
            w = s % 2

            @pl.when(c + 2 < _NCHUNK)
            def _():
                fire_gather(c + 2, (s + 2) % 4)

            wait_gather(s)

            @pl.when(c >= 2)
            def _():
                wait_write(w)

            compact(s, w)
            start_write(c, w)
        return carry

    lax.fori_loop(0, _NCHUNK // 4, body, 0)
    wait_write(0)
    wait_write(1)


def kernel(x, table):
    idx = x.T.reshape(-1).astype(jnp.int32)
    tabp = jnp.pad(table, ((0, 0), (0, 128 - _DIM)))
    out_t = _gather_kernel(idx, tabp)
    return out_t.reshape(_HIST, _BATCH, _DIM).transpose(1, 0, 2)


# speedup vs baseline: 1.0020x; 1.0020x over previous
import functools

import jax
import jax.numpy as jnp
from jax import lax
from jax.experimental import pallas as pl
from jax.experimental.pallas import tpu as pltpu
from jax.experimental.pallas import tpu_sc as plsc

_BATCH = 16384
_HIST = 50
_DIM = 64
_VOCAB = 1000000
_B = _BATCH * _HIST

_NC = 2
_NS = 16
_NW = _NC * _NS
_BPW = _B // _NW
_C = 128
_NCHUNK = _BPW // _C

_mesh = plsc.VectorSubcoreMesh(core_axis_name="c", subcore_axis_name="s")


@functools.partial(
    pl.kernel,
    mesh=_mesh,
    out_type=jax.ShapeDtypeStruct((_B, _DIM), jnp.float32),
    scratch_types=[
        pltpu.VMEM((_BPW,), jnp.int32),
        pltpu.VMEM((2, _C, 128), jnp.float32),
        pltpu.VMEM((2, _C, _DIM), jnp.float32),
    ]
    + [pltpu.SemaphoreType.DMA] * 4,
)
def _gather_kernel(idx_hbm, tabp_hbm, out_hbm, idx_all, pairs, rows,
                   gsem0, gsem1, wsem0, wsem1):
    wid = lax.axis_index("s") * _NC + lax.axis_index("c")
    base = wid * _BPW
    gsem = (gsem0, gsem1)
    wsem = (wsem0, wsem1)
    pltpu.sync_copy(idx_hbm.at[pl.ds(base, _BPW)], idx_all)

    def fire_gather(c, s):
        src = tabp_hbm.at[idx_all.at[pl.ds(c * _C, _C)]]
        pltpu.async_copy(src, pairs.at[s], gsem[s])

    def wait_gather(s):
        src = tabp_hbm.at[idx_all.at[pl.ds(0, _C)]]
        pltpu.make_async_copy(src, pairs.at[s], gsem[s]).wait()

    def compact(s, w):
        def gbody(g, carry):
            for u in range(16):
                j = g * 16 + u
                for k in range(_DIM // 16):
                    rows[w, j, pl.ds(k * 16, 16)] = (
                        pairs[s, j, pl.ds(k * 16, 16)]
                    )
            return carry

        lax.fori_loop(0, _C // 16, gbody, 0)

    def start_write(c, w):
        dst = out_hbm.at[pl.ds(base + c * _C, _C)]
        pltpu.async_copy(rows.at[w], dst, wsem[w])

    def wait_write(w):
        dst = out_hbm.at[pl.ds(base, _C)]
        pltpu.make_async_copy(rows.at[w], dst, wsem[w]).wait()

    fire_gather(0, 0)

    def body(i, carry):
        for s in range(2):
            c = 2 * i + s
            ns = 1 - s

            @pl.when(c + 1 < _NCHUNK)
            def _():
                fire_gather(c + 1, ns)

            wait_gather(s)

            @pl.when(c >= 2)
            def _():
                wait_write(s)

            compact(s, s)
            start_write(c, s)
        return carry

    lax.fori_loop(0, _NCHUNK // 2, body, 0)
    wait_write(0)
    wait_write(1)


def kernel(x, table):
    idx = x.T.reshape(-1).astype(jnp.int32)
    tabp = jnp.pad(table, ((0, 0), (0, 128 - _DIM)))
    out_t = _gather_kernel(idx, tabp)
    return out_t.reshape(_HIST, _BATCH, _DIM).transpose(1, 0, 2)
